# baseline (speedup 1.0000x reference)
import functools

import jax
import jax.numpy as jnp
from jax import lax
from jax.experimental import pallas as pl
from jax.experimental.pallas import tpu as pltpu
from jax.experimental.pallas import tpu_sc as plsc

F32 = jnp.float32
NC, NS = 2, 16


def _node_lin_body(h_ref, w_ref, b_ref, ah_ref, dh_ref, ehlo_ref, ehhi_ref):
    y = jnp.dot(h_ref[...], w_ref[...], preferred_element_type=F32) + b_ref[...]
    ah_ref[...] = y[:, :128]
    dh_ref[...] = y[:, 128:256]
    ehlo_ref[...] = y[:, 256:320]
    ehhi_ref[...] = y[:, 320:384]


def _node_lin(h, w_cat, b_cat):
    n = h.shape[0]
    bn = 2000
    return pl.pallas_call(
        _node_lin_body,
        grid=(n // bn,),
        in_specs=[
            pl.BlockSpec((bn, 128), lambda i: (i, 0)),
            pl.BlockSpec((128, 384), lambda i: (0, 0)),
            pl.BlockSpec((1, 384), lambda i: (0, 0)),
        ],
        out_specs=[
            pl.BlockSpec((bn, 128), lambda i: (i, 0)),
            pl.BlockSpec((bn, 128), lambda i: (i, 0)),
            pl.BlockSpec((bn, 64), lambda i: (i, 0)),
            pl.BlockSpec((bn, 64), lambda i: (i, 0)),
        ],
        out_shape=[
            jax.ShapeDtypeStruct((n, 128), F32),
            jax.ShapeDtypeStruct((n, 128), F32),
            jax.ShapeDtypeStruct((n, 64), F32),
            jax.ShapeDtypeStruct((n, 64), F32),
        ],
    )(h, w_cat, b_cat)


def _u_lin_body(u_ref, w_ref, b_ref, cu_ref, fu_ref):
    y = jnp.dot(u_ref[...], w_ref[...], preferred_element_type=F32) + b_ref[...]
    cu_ref[...] = y[:, :128]
    fu_ref[...] = y[:, 128:256]


def _u_lin(u, w_cat, b_cat):
    g = u.shape[0]
    return pl.pallas_call(
        _u_lin_body,
        out_shape=[
            jax.ShapeDtypeStruct((g, 128), F32),
            jax.ShapeDtypeStruct((g, 128), F32),
        ],
    )(u, w_cat, b_cat)


def _edge_body(e_ref, nb_ref, bg_ref, bwt_ref, bb_ref, cu_ref, eps_ref,
               epre_ref, stats_ref):
    i = pl.program_id(0)
    e = e_ref[...]
    be = jnp.dot(e, bwt_ref[...], preferred_element_type=F32) + bb_ref[...]
    ep = eps_ref[0, 0] + 1.0
    onehot = (bg_ref[...] == lax.broadcasted_iota(jnp.int32, (1, 16), 1)
              ).astype(F32)
    cu_rows = jnp.dot(onehot, cu_ref[...], preferred_element_type=F32)
    epb = ep * be
    x = (jnp.maximum(epb + nb_ref[...], 0.0)
         + jnp.maximum(epb + be, 0.0)
         + jnp.maximum(epb + cu_rows, 0.0))
    epre_ref[...] = x
    s1 = jnp.sum(x, axis=0, keepdims=True)
    s2 = jnp.sum(x * x, axis=0, keepdims=True)
    blk = jnp.concatenate([s1, s2], axis=0)

    @pl.when(i == 0)
    def _():
        stats_ref[...] = blk

    @pl.when(i > 0)
    def _():
        stats_ref[...] += blk


def _edge(e, neigh, bg2d, bwt, bb, cu, epsr):
    m = e.shape[0]
    bm = 1000
    return pl.pallas_call(
        _edge_body,
        grid=(m // bm,),
        in_specs=[
            pl.BlockSpec((bm, 128), lambda i: (i, 0)),
            pl.BlockSpec((bm, 128), lambda i: (i, 0)),
            pl.BlockSpec((bm, 1), lambda i: (i, 0)),
            pl.BlockSpec((128, 128), lambda i: (0, 0)),
            pl.BlockSpec((1, 128), lambda i: (0, 0)),
            pl.BlockSpec((16, 128), lambda i: (0, 0)),
            pl.BlockSpec((1, 1), lambda i: (0, 0)),
        ],
        out_specs=[
            pl.BlockSpec((bm, 128), lambda i: (i, 0)),
            pl.BlockSpec((2, 128), lambda i: (0, 0)),
        ],
        out_shape=[
            jax.ShapeDtypeStruct((m, 128), F32),
            jax.ShapeDtypeStruct((2, 128), F32),
        ],
    )(e, neigh, bg2d, bwt, bb, cu, epsr)


def _norm_body(x_ref, sc_ref, sh_ref, en_ref, siglo_ref, sighi_ref):
    en = jnp.maximum(x_ref[...] * sc_ref[...] + sh_ref[...], 0.0)
    en_ref[...] = en
    sig = 1.0 / (1.0 + jnp.exp(-en))
    siglo_ref[...] = sig[:, :64]
    sighi_ref[...] = sig[:, 64:]


def _norm(epre, scale, shift):
    m = epre.shape[0]
    bm = 1000
    return pl.pallas_call(
        _norm_body,
        grid=(m // bm,),
        in_specs=[
            pl.BlockSpec((bm, 128), lambda i: (i, 0)),
            pl.BlockSpec((1, 128), lambda i: (0, 0)),
            pl.BlockSpec((1, 128), lambda i: (0, 0)),
        ],
        out_specs=[
            pl.BlockSpec((bm, 128), lambda i: (i, 0)),
            pl.BlockSpec((bm, 64), lambda i: (i, 0)),
            pl.BlockSpec((bm, 64), lambda i: (i, 0)),
        ],
        out_shape=[
            jax.ShapeDtypeStruct((m, 128), F32),
            jax.ShapeDtypeStruct((m, 64), F32),
            jax.ShapeDtypeStruct((m, 64), F32),
        ],
    )(epre, scale, shift)


def _node_body(ah_ref, dh_ref, nh_ref, dh2_ref, ag_ref, fu_ref, eps_ref,
               g_ref, b_ref, out_ref):
    ah = ah_ref[...]
    ep = eps_ref[0, 0] + 1.0
    onehot = (ag_ref[...] == lax.broadcasted_iota(jnp.int32, (1, 16), 1)
              ).astype(F32)
    fu_rows = jnp.dot(onehot, fu_ref[...], preferred_element_type=F32)
    num = jnp.concatenate([nh_ref[0], nh_ref[1]], axis=-1)
    den = jnp.concatenate([dh2_ref[0], dh2_ref[1]], axis=-1)
    gated = num / (den + 1e-6)
    epa = ep * ah
    x = (jnp.maximum(epa + dh_ref[...], 0.0)
         + jnp.maximum(epa + fu_rows, 0.0)
         + gated)
    mu = jnp.mean(x, axis=0, keepdims=True)
    var = jnp.mean(x * x, axis=0, keepdims=True) - mu * mu
    y = (x - mu) * jax.lax.rsqrt(var + 1e-5) * g_ref[...] + b_ref[...]
    out_ref[...] = jnp.maximum(y, 0.0)


def _node(ah, dh, numh, denh, ag2d, fu, epsr, g, b):
    n = ah.shape[0]
    return pl.pallas_call(
        _node_body,
        out_shape=jax.ShapeDtypeStruct((n, 128), F32),
    )(ah, dh, numh, denh, ag2d, fu, epsr, g, b)


def _sc_neigh(ah, src, dst):
    m = src.shape[0]
    nw = NC * NS
    per_w = m // nw
    c = 104
    nb = 4
    n_rounds = per_w // (c * nb)
    tail = per_w - n_rounds * c * nb
    assert per_w * nw == m and c % 8 == 0 and tail % 8 == 0
    mesh = plsc.VectorSubcoreMesh(core_axis_name="c", subcore_axis_name="s")

    scratch = []
    for _ in range(nb):
        scratch += [pltpu.VMEM((c,), jnp.int32), pltpu.VMEM((c,), jnp.int32),
                    pltpu.VMEM((c, 128), F32)]
    scratch += [pltpu.SemaphoreType.DMA] * (4 * nb)

    @functools.partial(
        pl.kernel,
        out_type=jax.ShapeDtypeStruct((m, 128), F32),
        mesh=mesh,
        compiler_params=pltpu.CompilerParams(use_tc_tiling_on_sc=False),
        scratch_types=scratch,
    )
    def k(ah_hbm, src_hbm, dst_hbm, out_hbm, *scr):
        idx1 = [scr[3 * i] for i in range(nb)]
        idx2 = [scr[3 * i + 1] for i in range(nb)]
        rows = [scr[3 * i + 2] for i in range(nb)]
        sems = scr[3 * nb:]
        semld = sems[0:nb]
        semg1 = sems[nb:2 * nb]
        semg2 = sems[2 * nb:3 * nb]
        semst = sems[3 * nb:4 * nb]
        wid = lax.axis_index("s") * NC + lax.axis_index("c")
        base = wid * per_w

        def round_fn(r, carry):
            c0 = base + r * (c * nb)
            for i in range(nb):
                @pl.when(r > 0)
                def _(i=i):
                    pltpu.make_async_copy(
                        rows[i], out_hbm.at[pl.ds(0, c)], semst[i]).wait()
                pltpu.async_copy(
                    src_hbm.at[pl.ds(c0 + i * c, c)], idx1[i], semld[i])
                pltpu.async_copy(
                    dst_hbm.at[pl.ds(c0 + i * c, c)], idx2[i], semld[i])
            for i in range(nb):
                pltpu.make_async_copy(
                    src_hbm.at[pl.ds(0, c)], idx1[i], semld[i]).wait()
                pltpu.make_async_copy(
                    src_hbm.at[pl.ds(0, c)], idx2[i], semld[i]).wait()
                pltpu.async_copy(ah_hbm.at[idx1[i]], rows[i], semg1[i])
            for i in range(nb):
                pltpu.make_async_copy(
                    ah_hbm.at[idx1[i]], rows[i], semg1[i]).wait()
                pltpu.async_copy(ah_hbm.at[idx2[i]], rows[i], semg2[i],
                                 add=True)
            for i in range(nb):
                pltpu.make_async_copy(
                    ah_hbm.at[idx2[i]], rows[i], semg2[i]).wait()
                pltpu.async_copy(
                    rows[i], out_hbm.at[pl.ds(c0 + i * c, c)], semst[i])
            return carry

        lax.fori_loop(0, n_rounds, round_fn, 0)
        for i in range(nb):
            pltpu.make_async_copy(
                rows[i], out_hbm.at[pl.ds(0, c)], semst[i]).wait()
        if tail:
            tb = base + n_rounds * c * nb
            pltpu.sync_copy(src_hbm.at[pl.ds(tb, tail)],
                            idx1[0].at[pl.ds(0, tail)])
            pltpu.sync_copy(dst_hbm.at[pl.ds(tb, tail)],
                            idx2[0].at[pl.ds(0, tail)])
            rt = rows[0].at[pl.ds(0, tail)]
            pltpu.async_copy(ah_hbm.at[idx1[0].at[pl.ds(0, tail)]], rt,
                             semg1[0]).wait()
            pltpu.async_copy(ah_hbm.at[idx2[0].at[pl.ds(0, tail)]], rt,
                             semg2[0], add=True).wait()
            pltpu.sync_copy(rt, out_hbm.at[pl.ds(tb, tail)])

    return k(ah, src, dst)


def _sc_scatter(sigs_lo, sigs_hi, src, dst, ehs_lo, ehs_hi):
    m = src.shape[0]
    n = ehs_lo[0].shape[0]
    per_t = m // NS
    c = 64
    nb = 2
    n_rounds = per_t // (c * nb)
    tail = per_t - n_rounds * c * nb
    nz = (n // NS) & ~7
    rem = n - nz * NS
    assert per_t * NS == m and rem % 8 == 0 and tail % 8 == 0 and tail <= c
    mesh = plsc.VectorSubcoreMesh(core_axis_name="c", subcore_axis_name="s")

    scratch = []
    for _ in range(nb):
        scratch += [pltpu.VMEM((c,), jnp.int32), pltpu.VMEM((c,), jnp.int32),
                    pltpu.VMEM((c, 64), F32), pltpu.VMEM((c, 64), F32),
                    pltpu.VMEM((c, 64), F32), pltpu.VMEM((c, 64), F32),
                    pltpu.VMEM((c, 64), F32)]
    scratch += [pltpu.VMEM((16,), jnp.int32), pltpu.VMEM((16,), jnp.int32)]
    scratch += [pltpu.VMEM((80, 64), F32),
                pltpu.VMEM_SHARED((n, 64), F32),
                pltpu.VMEM_SHARED((n, 64), F32)]
    scratch += [pltpu.SemaphoreType.DMA] * (2 * nb)

    @functools.partial(
        pl.kernel,
        out_type=[jax.ShapeDtypeStruct((2, n, 64), F32) for _ in range(4)],
        mesh=mesh,
        compiler_params=pltpu.CompilerParams(use_tc_tiling_on_sc=False),
        scratch_types=scratch,
    )
    def k(siglo1_hbm, sighi1_hbm, siglo2_hbm, sighi2_hbm, src_hbm, dst_hbm,
          ehlo1_hbm, ehhi1_hbm, ehlo2_hbm, ehhi2_hbm,
          num1_hbm, den1_hbm, num2_hbm, den2_hbm, *scr):
        idx1 = [scr[7 * i] for i in range(nb)]
        idx2 = [scr[7 * i + 1] for i in range(nb)]
        sigb = [scr[7 * i + 2] for i in range(nb)]
        ehj = [scr[7 * i + 3] for i in range(nb)]
        ehi = [scr[7 * i + 4] for i in range(nb)]
        payi = [scr[7 * i + 5] for i in range(nb)]
        payj = [scr[7 * i + 6] for i in range(nb)]
        p = 7 * nb
        idxt1, idxt2, zb, accn, accd = scr[p:p + 5]
        sems = scr[p + 5:]
        semld = sems[0:nb]
        semg = sems[nb:2 * nb]
        cid = lax.axis_index("c")
        sid = lax.axis_index("s")
        r0 = sid * nz
        last = sid == NS - 1

        def zrow(r, carry):
            for kk in range(4):
                zb[r, pl.ds(16 * kk, 16)] = jnp.zeros((16,), F32)
            return carry

        lax.fori_loop(0, 80, zrow, 0)

        def zero_rows(dst_ref, total):
            off = 0
            while total > 0:
                cnt = min(80, total)
                pltpu.sync_copy(zb.at[pl.ds(0, cnt)],
                                dst_ref.at[pl.ds(r0 + off, cnt)])
                off += cnt
                total -= cnt

        def mulrows(i, nrows, roff):
            def row(r, carry2):
                for kk in range(4):
                    sl = pl.ds(16 * kk, 16)
                    s = sigb[i][r, sl]
                    payi[i][r, sl] = s * ehj[i][r, sl]
                    payj[i][r, sl] = s * ehi[i][r, sl]
                return carry2
            lax.fori_loop(roff, roff + nrows, row, 0, unroll=4)

        def phase(sig_hbm, eh_hbm, num_out, den_out):
            plsc.subcore_barrier()
            zero_rows(accn, nz + rem)
            zero_rows(accd, nz + rem)
            plsc.subcore_barrier()
            base = sid * per_t

            def round_fn(r, carry):
                c0 = base + r * (c * nb)
                for i in range(nb):
                    pltpu.async_copy(
                        src_hbm.at[pl.ds(c0 + i * c, c)], idx1[i], semld[i])
                    pltpu.async_copy(
                        dst_hbm.at[pl.ds(c0 + i * c, c)], idx2[i], semld[i])
                    pltpu.async_copy(
                        sig_hbm.at[pl.ds(c0 + i * c, c)], sigb[i], semld[i])
                for i in range(nb):
                    pltpu.make_async_copy(
                        src_hbm.at[pl.ds(0, c)], idx1[i], semld[i]).wait()
                    pltpu.make_async_copy(
                        src_hbm.at[pl.ds(0, c)], idx2[i], semld[i]).wait()
                    pltpu.make_async_copy(
                        sig_hbm.at[pl.ds(0, c)], sigb[i], semld[i]).wait()
                    pltpu.async_copy(eh_hbm.at[idx2[i]], ehj[i], semg[i])
                    pltpu.async_copy(eh_hbm.at[idx1[i]], ehi[i], semg[i])
                for i in range(nb):
                    pltpu.make_async_copy(
                        eh_hbm.at[idx2[i]], ehj[i], semg[i]).wait()
                    pltpu.make_async_copy(
                        eh_hbm.at[idx1[i]], ehi[i], semg[i]).wait()
                    mulrows(i, c, 0)
                    pltpu.sync_copy(payi[i], accn.at[idx1[i]], add=True)
                    pltpu.sync_copy(payj[i], accn.at[idx2[i]], add=True)
                    pltpu.sync_copy(sigb[i], accd.at[idx1[i]], add=True)
                    pltpu.sync_copy(sigb[i], accd.at[idx2[i]], add=True)
                return carry

            lax.fori_loop(0, n_rounds, round_fn, 0)
            if tail:
                tb = base + n_rounds * c * nb
                pltpu.sync_copy(src_hbm.at[pl.ds(tb, tail)], idxt1)
                pltpu.sync_copy(dst_hbm.at[pl.ds(tb, tail)], idxt2)
                pltpu.sync_copy(sig_hbm.at[pl.ds(tb, tail)],
                                sigb[0].at[pl.ds(0, tail)])
                pltpu.async_copy(eh_hbm.at[idxt2],
                                 ehj[0].at[pl.ds(0, tail)], semg[0]).wait()
                pltpu.async_copy(eh_hbm.at[idxt1],
                                 ehi[0].at[pl.ds(0, tail)], semg[0]).wait()
                mulrows(0, tail, 0)
                pltpu.sync_copy(payi[0].at[pl.ds(0, tail)],
                                accn.at[idxt1], add=True)
                pltpu.sync_copy(payj[0].at[pl.ds(0, tail)],
                                accn.at[idxt2], add=True)
                pltpu.sync_copy(sigb[0].at[pl.ds(0, tail)],
                                accd.at[idxt1], add=True)
                pltpu.sync_copy(sigb[0].at[pl.ds(0, tail)],
                                accd.at[idxt2], add=True)
            plsc.subcore_barrier()

            def dump(cnt):
                pltpu.sync_copy(accn.at[pl.ds(r0, cnt)],
                                num_out.at[pl.ds(r0, cnt)])
                pltpu.sync_copy(accd.at[pl.ds(r0, cnt)],
                                den_out.at[pl.ds(r0, cnt)])

            @pl.when(jnp.logical_not(last))
            def _():
                dump(nz)

            @pl.when(last)
            def _():
                dump(nz + rem)

        @pl.when(cid == 0)
        def _():
            phase(siglo1_hbm, ehlo1_hbm, num1_hbm.at[0], den1_hbm.at[0])
            phase(siglo2_hbm, ehlo2_hbm, num2_hbm.at[0], den2_hbm.at[0])

        @pl.when(cid == 1)
        def _():
            phase(sighi1_hbm, ehhi1_hbm, num1_hbm.at[1], den1_hbm.at[1])
            phase(sighi2_hbm, ehhi2_hbm, num2_hbm.at[1], den2_hbm.at[1])

    return k(sigs_lo[0], sigs_hi[0], sigs_lo[1], sigs_hi[1], src, dst,
             ehs_lo[0], ehs_hi[0], ehs_lo[1], ehs_hi[1])


def kernel(h_atom, e_bond, h_atom2, e_bond2, u_global, bond_atoms, bond_graph,
           atom_graph, eps, A_W, A_b, B_W, B_b, C_W, C_b, D_W, D_b, E_W, E_b,
           F_W, F_b, bn_e_g, bn_e_b, bn_h_g, bn_h_b):
    m = e_bond.shape[0]
    src = bond_atoms[:, 0]
    dst = bond_atoms[:, 1]
    bg2d = bond_graph[:, None]
    ag2d = atom_graph[:, None]
    epsr = eps.reshape(1, 1)

    w_node = jnp.concatenate([A_W.T, D_W.T, E_W.T], axis=1)
    b_node = jnp.concatenate([A_b, D_b, E_b])[None, :]
    w_u = jnp.concatenate([C_W.T, F_W.T], axis=1)
    b_u = jnp.concatenate([C_b, F_b])[None, :]
    bwt = B_W.T
    bb = B_b[None, :]

    cu, fu = _u_lin(u_global, w_u, b_u)

    def edge_stage(h, e):
        ah, dh, ehlo, ehhi = _node_lin(h, w_node, b_node)
        neigh = _sc_neigh(ah, src, dst)
        epre, stats = _edge(e, neigh, bg2d, bwt, bb, cu, epsr)
        mu = stats[0] / m
        var = stats[1] / m - mu * mu
        scale = bn_e_g * lax.rsqrt(var + 1e-5)
        shift = bn_e_b - mu * scale
        e_new, siglo, sighi = _norm(epre, scale[None, :], shift[None, :])
        return ah, dh, ehlo, ehhi, e_new, siglo, sighi

    ah1, dh1, ehlo1, ehhi1, e1, siglo1, sighi1 = edge_stage(h_atom, e_bond)
    ah2, dh2, ehlo2, ehhi2, e2, siglo2, sighi2 = edge_stage(h_atom2, e_bond2)

    numh1, denh1, numh2, denh2 = _sc_scatter(
        (siglo1, siglo2), (sighi1, sighi2), src, dst,
        (ehlo1, ehlo2), (ehhi1, ehhi2))

    h1 = _node(ah1, dh1, numh1, denh1, ag2d, fu, epsr,
               bn_h_g[None, :], bn_h_b[None, :])
    h2 = _node(ah2, dh2, numh2, denh2, ag2d, fu, epsr,
               bn_h_g[None, :], bn_h_b[None, :])
    return (h1, e1, h2, e2)


# speedup vs baseline: 1.0163x; 1.0163x over previous
import functools

import jax
import jax.numpy as jnp
from jax import lax
from jax.experimental import pallas as pl
from jax.experimental.pallas import tpu as pltpu
from jax.experimental.pallas import tpu_sc as plsc

F32 = jnp.float32
NC, NS = 2, 16


def _node_lin_body(h_ref, w_ref, b_ref, ah_ref, dh_ref, ehlo_ref, ehhi_ref):
    y = jnp.dot(h_ref[...], w_ref[...], preferred_element_type=F32) + b_ref[...]
    ah_ref[...] = y[:, :128]
    dh_ref[...] = y[:, 128:256]
    ehlo_ref[...] = y[:, 256:320]
    ehhi_ref[...] = y[:, 320:384]


def _node_lin(h, w_cat, b_cat):
    n = h.shape[0]
    bn = 2000
    return pl.pallas_call(
        _node_lin_body,
        grid=(n // bn,),
        in_specs=[
            pl.BlockSpec((bn, 128), lambda i: (i, 0)),
            pl.BlockSpec((128, 384), lambda i: (0, 0)),
            pl.BlockSpec((1, 384), lambda i: (0, 0)),
        ],
        out_specs=[
            pl.BlockSpec((bn, 128), lambda i: (i, 0)),
            pl.BlockSpec((bn, 128), lambda i: (i, 0)),
            pl.BlockSpec((bn, 64), lambda i: (i, 0)),
            pl.BlockSpec((bn, 64), lambda i: (i, 0)),
        ],
        out_shape=[
            jax.ShapeDtypeStruct((n, 128), F32),
            jax.ShapeDtypeStruct((n, 128), F32),
            jax.ShapeDtypeStruct((n, 64), F32),
            jax.ShapeDtypeStruct((n, 64), F32),
        ],
    )(h, w_cat, b_cat)


def _u_lin_body(u_ref, w_ref, b_ref, cu_ref, fu_ref):
    y = jnp.dot(u_ref[...], w_ref[...], preferred_element_type=F32) + b_ref[...]
    cu_ref[...] = y[:, :128]
    fu_ref[...] = y[:, 128:256]


def _u_lin(u, w_cat, b_cat):
    g = u.shape[0]
    return pl.pallas_call(
        _u_lin_body,
        out_shape=[
            jax.ShapeDtypeStruct((g, 128), F32),
            jax.ShapeDtypeStruct((g, 128), F32),
        ],
    )(u, w_cat, b_cat)


def _edge_body(e_ref, nb_ref, bg_ref, bwt_ref, bb_ref, cu_ref, eps_ref,
               epre_ref, stats_ref):
    i = pl.program_id(0)
    e = e_ref[...]
    be = jnp.dot(e, bwt_ref[...], preferred_element_type=F32) + bb_ref[...]
    ep = eps_ref[0, 0] + 1.0
    onehot = (bg_ref[...] == lax.broadcasted_iota(jnp.int32, (1, 16), 1)
              ).astype(F32)
    cu_rows = jnp.dot(onehot, cu_ref[...], preferred_element_type=F32)
    epb = ep * be
    x = (jnp.maximum(epb + nb_ref[...], 0.0)
         + jnp.maximum(epb + be, 0.0)
         + jnp.maximum(epb + cu_rows, 0.0))
    epre_ref[...] = x
    s1 = jnp.sum(x, axis=0, keepdims=True)
    s2 = jnp.sum(x * x, axis=0, keepdims=True)
    blk = jnp.concatenate([s1, s2], axis=0)

    @pl.when(i == 0)
    def _():
        stats_ref[...] = blk

    @pl.when(i > 0)
    def _():
        stats_ref[...] += blk


def _edge(e, neigh, bg2d, bwt, bb, cu, epsr):
    m = e.shape[0]
    bm = 1000
    return pl.pallas_call(
        _edge_body,
        grid=(m // bm,),
        in_specs=[
            pl.BlockSpec((bm, 128), lambda i: (i, 0)),
            pl.BlockSpec((bm, 128), lambda i: (i, 0)),
            pl.BlockSpec((bm, 1), lambda i: (i, 0)),
            pl.BlockSpec((128, 128), lambda i: (0, 0)),
            pl.BlockSpec((1, 128), lambda i: (0, 0)),
            pl.BlockSpec((16, 128), lambda i: (0, 0)),
            pl.BlockSpec((1, 1), lambda i: (0, 0)),
        ],
        out_specs=[
            pl.BlockSpec((bm, 128), lambda i: (i, 0)),
            pl.BlockSpec((2, 128), lambda i: (0, 0)),
        ],
        out_shape=[
            jax.ShapeDtypeStruct((m, 128), F32),
            jax.ShapeDtypeStruct((2, 128), F32),
        ],
    )(e, neigh, bg2d, bwt, bb, cu, epsr)


def _norm_body(x_ref, sc_ref, sh_ref, en_ref, siglo_ref, sighi_ref):
    en = jnp.maximum(x_ref[...] * sc_ref[...] + sh_ref[...], 0.0)
    en_ref[...] = en
    sig = 1.0 / (1.0 + jnp.exp(-en))
    siglo_ref[...] = sig[:, :64]
    sighi_ref[...] = sig[:, 64:]


def _norm(epre, scale, shift):
    m = epre.shape[0]
    bm = 1000
    return pl.pallas_call(
        _norm_body,
        grid=(m // bm,),
        in_specs=[
            pl.BlockSpec((bm, 128), lambda i: (i, 0)),
            pl.BlockSpec((1, 128), lambda i: (0, 0)),
            pl.BlockSpec((1, 128), lambda i: (0, 0)),
        ],
        out_specs=[
            pl.BlockSpec((bm, 128), lambda i: (i, 0)),
            pl.BlockSpec((bm, 64), lambda i: (i, 0)),
            pl.BlockSpec((bm, 64), lambda i: (i, 0)),
        ],
        out_shape=[
            jax.ShapeDtypeStruct((m, 128), F32),
            jax.ShapeDtypeStruct((m, 64), F32),
            jax.ShapeDtypeStruct((m, 64), F32),
        ],
    )(epre, scale, shift)


def _node_body(ah_ref, dh_ref, nh_ref, dh2_ref, ag_ref, fu_ref, eps_ref,
               g_ref, b_ref, out_ref):
    ah = ah_ref[...]
    ep = eps_ref[0, 0] + 1.0
    onehot = (ag_ref[...] == lax.broadcasted_iota(jnp.int32, (1, 16), 1)
              ).astype(F32)
    fu_rows = jnp.dot(onehot, fu_ref[...], preferred_element_type=F32)
    num = jnp.concatenate([nh_ref[0], nh_ref[1]], axis=-1)
    den = jnp.concatenate([dh2_ref[0], dh2_ref[1]], axis=-1)
    gated = num / (den + 1e-6)
    epa = ep * ah
    x = (jnp.maximum(epa + dh_ref[...], 0.0)
         + jnp.maximum(epa + fu_rows, 0.0)
         + gated)
    mu = jnp.mean(x, axis=0, keepdims=True)
    var = jnp.mean(x * x, axis=0, keepdims=True) - mu * mu
    y = (x - mu) * jax.lax.rsqrt(var + 1e-5) * g_ref[...] + b_ref[...]
    out_ref[...] = jnp.maximum(y, 0.0)


def _node(ah, dh, numh, denh, ag2d, fu, epsr, g, b):
    n = ah.shape[0]
    return pl.pallas_call(
        _node_body,
        out_shape=jax.ShapeDtypeStruct((n, 128), F32),
    )(ah, dh, numh, denh, ag2d, fu, epsr, g, b)


def _sc_neigh(ah, src, dst):
    m = src.shape[0]
    nw = NC * NS
    per_w = m // nw
    c = 104
    nb = 4
    n_rounds = per_w // (c * nb)
    tail = per_w - n_rounds * c * nb
    assert per_w * nw == m and c % 8 == 0 and tail % 8 == 0
    mesh = plsc.VectorSubcoreMesh(core_axis_name="c", subcore_axis_name="s")

    scratch = []
    for _ in range(nb):
        scratch += [pltpu.VMEM((c,), jnp.int32), pltpu.VMEM((c,), jnp.int32),
                    pltpu.VMEM((c, 128), F32)]
    scratch += [pltpu.SemaphoreType.DMA] * (4 * nb)

    @functools.partial(
        pl.kernel,
        out_type=jax.ShapeDtypeStruct((m, 128), F32),
        mesh=mesh,
        compiler_params=pltpu.CompilerParams(use_tc_tiling_on_sc=False),
        scratch_types=scratch,
    )
    def k(ah_hbm, src_hbm, dst_hbm, out_hbm, *scr):
        idx1 = [scr[3 * i] for i in range(nb)]
        idx2 = [scr[3 * i + 1] for i in range(nb)]
        rows = [scr[3 * i + 2] for i in range(nb)]
        sems = scr[3 * nb:]
        semld = sems[0:nb]
        semg1 = sems[nb:2 * nb]
        semg2 = sems[2 * nb:3 * nb]
        semst = sems[3 * nb:4 * nb]
        wid = lax.axis_index("s") * NC + lax.axis_index("c")
        base = wid * per_w

        def round_fn(r, carry):
            c0 = base + r * (c * nb)
            for i in range(nb):
                @pl.when(r > 0)
                def _(i=i):
                    pltpu.make_async_copy(
                        rows[i], out_hbm.at[pl.ds(0, c)], semst[i]).wait()
                pltpu.async_copy(
                    src_hbm.at[pl.ds(c0 + i * c, c)], idx1[i], semld[i])
                pltpu.async_copy(
                    dst_hbm.at[pl.ds(c0 + i * c, c)], idx2[i], semld[i])
            for i in range(nb):
                pltpu.make_async_copy(
                    src_hbm.at[pl.ds(0, c)], idx1[i], semld[i]).wait()
                pltpu.make_async_copy(
                    src_hbm.at[pl.ds(0, c)], idx2[i], semld[i]).wait()
                pltpu.async_copy(ah_hbm.at[idx1[i]], rows[i], semg1[i])
            for i in range(nb):
                pltpu.make_async_copy(
                    ah_hbm.at[idx1[i]], rows[i], semg1[i]).wait()
                pltpu.async_copy(ah_hbm.at[idx2[i]], rows[i], semg2[i],
                                 add=True)
            for i in range(nb):
                pltpu.make_async_copy(
                    ah_hbm.at[idx2[i]], rows[i], semg2[i]).wait()
                pltpu.async_copy(
                    rows[i], out_hbm.at[pl.ds(c0 + i * c, c)], semst[i])
            return carry

        lax.fori_loop(0, n_rounds, round_fn, 0)
        for i in range(nb):
            pltpu.make_async_copy(
                rows[i], out_hbm.at[pl.ds(0, c)], semst[i]).wait()
        if tail:
            tb = base + n_rounds * c * nb
            pltpu.sync_copy(src_hbm.at[pl.ds(tb, tail)],
                            idx1[0].at[pl.ds(0, tail)])
            pltpu.sync_copy(dst_hbm.at[pl.ds(tb, tail)],
                            idx2[0].at[pl.ds(0, tail)])
            rt = rows[0].at[pl.ds(0, tail)]
            pltpu.async_copy(ah_hbm.at[idx1[0].at[pl.ds(0, tail)]], rt,
                             semg1[0]).wait()
            pltpu.async_copy(ah_hbm.at[idx2[0].at[pl.ds(0, tail)]], rt,
                             semg2[0], add=True).wait()
            pltpu.sync_copy(rt, out_hbm.at[pl.ds(tb, tail)])

    return k(ah, src, dst)


def _sc_scatter(sigs_lo, sigs_hi, src, dst, ehs_lo, ehs_hi):
    m = src.shape[0]
    n = ehs_lo[0].shape[0]
    per_t = m // NS
    c = 64
    nb = 2
    n_rounds = per_t // (c * nb)
    tail = per_t - n_rounds * c * nb
    nz = (n // NS) & ~7
    rem = n - nz * NS
    assert per_t * NS == m and rem % 8 == 0 and tail % 8 == 0 and tail <= c
    mesh = plsc.VectorSubcoreMesh(core_axis_name="c", subcore_axis_name="s")

    scratch = []
    for _ in range(nb):
        scratch += [pltpu.VMEM((c,), jnp.int32), pltpu.VMEM((c,), jnp.int32),
                    pltpu.VMEM((c, 64), F32), pltpu.VMEM((c, 64), F32),
                    pltpu.VMEM((c, 64), F32), pltpu.VMEM((c, 64), F32),
                    pltpu.VMEM((c, 64), F32)]
    scratch += [pltpu.VMEM((16,), jnp.int32), pltpu.VMEM((16,), jnp.int32)]
    scratch += [pltpu.VMEM((80, 64), F32),
                pltpu.VMEM_SHARED((n, 64), F32),
                pltpu.VMEM_SHARED((n, 64), F32)]
    scratch += [pltpu.SemaphoreType.DMA] * (3 * nb)

    @functools.partial(
        pl.kernel,
        out_type=[jax.ShapeDtypeStruct((2, n, 64), F32) for _ in range(4)],
        mesh=mesh,
        compiler_params=pltpu.CompilerParams(use_tc_tiling_on_sc=False),
        scratch_types=scratch,
    )
    def k(siglo1_hbm, sighi1_hbm, siglo2_hbm, sighi2_hbm, src_hbm, dst_hbm,
          ehlo1_hbm, ehhi1_hbm, ehlo2_hbm, ehhi2_hbm,
          num1_hbm, den1_hbm, num2_hbm, den2_hbm, *scr):
        idx1 = [scr[7 * i] for i in range(nb)]
        idx2 = [scr[7 * i + 1] for i in range(nb)]
        sigb = [scr[7 * i + 2] for i in range(nb)]
        ehj = [scr[7 * i + 3] for i in range(nb)]
        ehi = [scr[7 * i + 4] for i in range(nb)]
        payi = [scr[7 * i + 5] for i in range(nb)]
        payj = [scr[7 * i + 6] for i in range(nb)]
        p = 7 * nb
        idxt1, idxt2, zb, accn, accd = scr[p:p + 5]
        sems = scr[p + 5:]
        semld = sems[0:nb]
        semg = sems[nb:2 * nb]
        semsc = sems[2 * nb:3 * nb]
        cid = lax.axis_index("c")
        sid = lax.axis_index("s")
        r0 = sid * nz
        last = sid == NS - 1

        def zrow(r, carry):
            for kk in range(4):
                zb[r, pl.ds(16 * kk, 16)] = jnp.zeros((16,), F32)
            return carry

        lax.fori_loop(0, 80, zrow, 0)

        def zero_rows(dst_ref, total):
            off = 0
            while total > 0:
                cnt = min(80, total)
                pltpu.sync_copy(zb.at[pl.ds(0, cnt)],
                                dst_ref.at[pl.ds(r0 + off, cnt)])
                off += cnt
                total -= cnt

        def mulrows(i, nrows, roff):
            def row(r, carry2):
                for kk in range(4):
                    sl = pl.ds(16 * kk, 16)
                    s = sigb[i][r, sl]
                    payi[i][r, sl] = s * ehj[i][r, sl]
                    payj[i][r, sl] = s * ehi[i][r, sl]
                return carry2
            lax.fori_loop(roff, roff + nrows, row, 0, unroll=4)

        def phase(sig_hbm, eh_hbm, num_out, den_out):
            plsc.subcore_barrier()
            zero_rows(accn, nz + rem)
            zero_rows(accd, nz + rem)
            plsc.subcore_barrier()
            base = sid * per_t

            def round_fn(r, carry):
                c0 = base + r * (c * nb)
                for i in range(nb):
                    pltpu.async_copy(
                        src_hbm.at[pl.ds(c0 + i * c, c)], idx1[i], semld[i])
                    pltpu.async_copy(
                        dst_hbm.at[pl.ds(c0 + i * c, c)], idx2[i], semld[i])
                    pltpu.async_copy(
                        sig_hbm.at[pl.ds(c0 + i * c, c)], sigb[i], semld[i])
                for i in range(nb):
                    pltpu.make_async_copy(
                        src_hbm.at[pl.ds(0, c)], idx1[i], semld[i]).wait()
                    pltpu.make_async_copy(
                        src_hbm.at[pl.ds(0, c)], idx2[i], semld[i]).wait()
                    pltpu.make_async_copy(
                        sig_hbm.at[pl.ds(0, c)], sigb[i], semld[i]).wait()
                    pltpu.async_copy(eh_hbm.at[idx2[i]], ehj[i], semg[i])
                    pltpu.async_copy(eh_hbm.at[idx1[i]], ehi[i], semg[i])
                for i in range(nb):
                    pltpu.make_async_copy(
                        eh_hbm.at[idx2[i]], ehj[i], semg[i]).wait()
                    pltpu.make_async_copy(
                        eh_hbm.at[idx1[i]], ehi[i], semg[i]).wait()
                    mulrows(i, c, 0)
                    pltpu.async_copy(payi[i], accn.at[idx1[i]], semsc[i],
                                     add=True)
                    pltpu.async_copy(sigb[i], accd.at[idx2[i]], semsc[i],
                                     add=True)
                    pltpu.make_async_copy(
                        payi[i], accn.at[idx1[i]], semsc[i]).wait()
                    pltpu.make_async_copy(
                        sigb[i], accd.at[idx2[i]], semsc[i]).wait()
                    pltpu.async_copy(payj[i], accn.at[idx2[i]], semsc[i],
                                     add=True)
                    pltpu.async_copy(sigb[i], accd.at[idx1[i]], semsc[i],
                                     add=True)
                    pltpu.make_async_copy(
                        payj[i], accn.at[idx2[i]], semsc[i]).wait()
                    pltpu.make_async_copy(
                        sigb[i], accd.at[idx1[i]], semsc[i]).wait()
                return carry

            lax.fori_loop(0, n_rounds, round_fn, 0)
            if tail:
                tb = base + n_rounds * c * nb
                pltpu.sync_copy(src_hbm.at[pl.ds(tb, tail)], idxt1)
                pltpu.sync_copy(dst_hbm.at[pl.ds(tb, tail)], idxt2)
                pltpu.sync_copy(sig_hbm.at[pl.ds(tb, tail)],
                                sigb[0].at[pl.ds(0, tail)])
                pltpu.async_copy(eh_hbm.at[idxt2],
                                 ehj[0].at[pl.ds(0, tail)], semg[0]).wait()
                pltpu.async_copy(eh_hbm.at[idxt1],
                                 ehi[0].at[pl.ds(0, tail)], semg[0]).wait()
                mulrows(0, tail, 0)
                pltpu.sync_copy(payi[0].at[pl.ds(0, tail)],
                                accn.at[idxt1], add=True)
                pltpu.sync_copy(payj[0].at[pl.ds(0, tail)],
                                accn.at[idxt2], add=True)
                pltpu.sync_copy(sigb[0].at[pl.ds(0, tail)],
                                accd.at[idxt1], add=True)
                pltpu.sync_copy(sigb[0].at[pl.ds(0, tail)],
                                accd.at[idxt2], add=True)
            plsc.subcore_barrier()

            def dump(cnt):
                pltpu.sync_copy(accn.at[pl.ds(r0, cnt)],
                                num_out.at[pl.ds(r0, cnt)])
                pltpu.sync_copy(accd.at[pl.ds(r0, cnt)],
                                den_out.at[pl.ds(r0, cnt)])

            @pl.when(jnp.logical_not(last))
            def _():
                dump(nz)

            @pl.when(last)
            def _():
                dump(nz + rem)

        @pl.when(cid == 0)
        def _():
            phase(siglo1_hbm, ehlo1_hbm, num1_hbm.at[0], den1_hbm.at[0])
            phase(siglo2_hbm, ehlo2_hbm, num2_hbm.at[0], den2_hbm.at[0])

        @pl.when(cid == 1)
        def _():
            phase(sighi1_hbm, ehhi1_hbm, num1_hbm.at[1], den1_hbm.at[1])
            phase(sighi2_hbm, ehhi2_hbm, num2_hbm.at[1], den2_hbm.at[1])

    return k(sigs_lo[0], sigs_hi[0], sigs_lo[1], sigs_hi[1], src, dst,
             ehs_lo[0], ehs_hi[0], ehs_lo[1], ehs_hi[1])


def kernel(h_atom, e_bond, h_atom2, e_bond2, u_global, bond_atoms, bond_graph,
           atom_graph, eps, A_W, A_b, B_W, B_b, C_W, C_b, D_W, D_b, E_W, E_b,
           F_W, F_b, bn_e_g, bn_e_b, bn_h_g, bn_h_b):
    m = e_bond.shape[0]
    src = bond_atoms[:, 0]
    dst = bond_atoms[:, 1]
    bg2d = bond_graph[:, None]
    ag2d = atom_graph[:, None]
    epsr = eps.reshape(1, 1)

    w_node = jnp.concatenate([A_W.T, D_W.T, E_W.T], axis=1)
    b_node = jnp.concatenate([A_b, D_b, E_b])[None, :]
    w_u = jnp.concatenate([C_W.T, F_W.T], axis=1)
    b_u = jnp.concatenate([C_b, F_b])[None, :]
    bwt = B_W.T
    bb = B_b[None, :]

    cu, fu = _u_lin(u_global, w_u, b_u)

    def edge_stage(h, e):
        ah, dh, ehlo, ehhi = _node_lin(h, w_node, b_node)
        neigh = _sc_neigh(ah, src, dst)
        epre, stats = _edge(e, neigh, bg2d, bwt, bb, cu, epsr)
        mu = stats[0] / m
        var = stats[1] / m - mu * mu
        scale = bn_e_g * lax.rsqrt(var + 1e-5)
        shift = bn_e_b - mu * scale
        e_new, siglo, sighi = _norm(epre, scale[None, :], shift[None, :])
        return ah, dh, ehlo, ehhi, e_new, siglo, sighi

    ah1, dh1, ehlo1, ehhi1, e1, siglo1, sighi1 = edge_stage(h_atom, e_bond)
    ah2, dh2, ehlo2, ehhi2, e2, siglo2, sighi2 = edge_stage(h_atom2, e_bond2)

    numh1, denh1, numh2, denh2 = _sc_scatter(
        (siglo1, siglo2), (sighi1, sighi2), src, dst,
        (ehlo1, ehlo2), (ehhi1, ehhi2))

    h1 = _node(ah1, dh1, numh1, denh1, ag2d, fu, epsr,
               bn_h_g[None, :], bn_h_b[None, :])
    h2 = _node(ah2, dh2, numh2, denh2, ag2d, fu, epsr,
               bn_h_g[None, :], bn_h_b[None, :])
    return (h1, e1, h2, e2)


# speedup vs baseline: 1.0204x; 1.0040x over previous
import functools

import jax
import jax.numpy as jnp
from jax import lax
from jax.experimental import pallas as pl
from jax.experimental.pallas import tpu as pltpu
from jax.experimental.pallas import tpu_sc as plsc

F32 = jnp.float32
NC, NS = 2, 16


def _node_lin_body(h_ref, w_ref, b_ref, ah_ref, dh_ref, ehlo_ref, ehhi_ref):
    y = jnp.dot(h_ref[...], w_ref[...], preferred_element_type=F32) + b_ref[...]
    ah_ref[...] = y[:, :128]
    dh_ref[...] = y[:, 128:256]
    ehlo_ref[...] = y[:, 256:320]
    ehhi_ref[...] = y[:, 320:384]


def _node_lin(h, w_cat, b_cat):
    n = h.shape[0]
    bn = 2000
    return pl.pallas_call(
        _node_lin_body,
        grid=(n // bn,),
        in_specs=[
            pl.BlockSpec((bn, 128), lambda i: (i, 0)),
            pl.BlockSpec((128, 384), lambda i: (0, 0)),
            pl.BlockSpec((1, 384), lambda i: (0, 0)),
        ],
        out_specs=[
            pl.BlockSpec((bn, 128), lambda i: (i, 0)),
            pl.BlockSpec((bn, 128), lambda i: (i, 0)),
            pl.BlockSpec((bn, 64), lambda i: (i, 0)),
            pl.BlockSpec((bn, 64), lambda i: (i, 0)),
        ],
        out_shape=[
            jax.ShapeDtypeStruct((n, 128), F32),
            jax.ShapeDtypeStruct((n, 128), F32),
            jax.ShapeDtypeStruct((n, 64), F32),
            jax.ShapeDtypeStruct((n, 64), F32),
        ],
    )(h, w_cat, b_cat)


def _u_lin_body(u_ref, w_ref, b_ref, cu_ref, fu_ref):
    y = jnp.dot(u_ref[...], w_ref[...], preferred_element_type=F32) + b_ref[...]
    cu_ref[...] = y[:, :128]
    fu_ref[...] = y[:, 128:256]


def _u_lin(u, w_cat, b_cat):
    g = u.shape[0]
    return pl.pallas_call(
        _u_lin_body,
        out_shape=[
            jax.ShapeDtypeStruct((g, 128), F32),
            jax.ShapeDtypeStruct((g, 128), F32),
        ],
    )(u, w_cat, b_cat)


def _edge_body(e_ref, nb_ref, bg_ref, bwt_ref, bb_ref, cu_ref, eps_ref,
               epre_ref, stats_ref):
    i = pl.program_id(0)
    e = e_ref[...]
    be = jnp.dot(e, bwt_ref[...], preferred_element_type=F32) + bb_ref[...]
    ep = eps_ref[0, 0] + 1.0
    onehot = (bg_ref[...] == lax.broadcasted_iota(jnp.int32, (1, 16), 1)
              ).astype(F32)
    cu_rows = jnp.dot(onehot, cu_ref[...], preferred_element_type=F32)
    epb = ep * be
    x = (jnp.maximum(epb + nb_ref[...], 0.0)
         + jnp.maximum(epb + be, 0.0)
         + jnp.maximum(epb + cu_rows, 0.0))
    epre_ref[...] = x
    s1 = jnp.sum(x, axis=0, keepdims=True)
    s2 = jnp.sum(x * x, axis=0, keepdims=True)
    blk = jnp.concatenate([s1, s2], axis=0)

    @pl.when(i == 0)
    def _():
        stats_ref[...] = blk

    @pl.when(i > 0)
    def _():
        stats_ref[...] += blk


def _edge(e, neigh, bg2d, bwt, bb, cu, epsr):
    m = e.shape[0]
    bm = 1000
    return pl.pallas_call(
        _edge_body,
        grid=(m // bm,),
        in_specs=[
            pl.BlockSpec((bm, 128), lambda i: (i, 0)),
            pl.BlockSpec((bm, 128), lambda i: (i, 0)),
            pl.BlockSpec((bm, 1), lambda i: (i, 0)),
            pl.BlockSpec((128, 128), lambda i: (0, 0)),
            pl.BlockSpec((1, 128), lambda i: (0, 0)),
            pl.BlockSpec((16, 128), lambda i: (0, 0)),
            pl.BlockSpec((1, 1), lambda i: (0, 0)),
        ],
        out_specs=[
            pl.BlockSpec((bm, 128), lambda i: (i, 0)),
            pl.BlockSpec((2, 128), lambda i: (0, 0)),
        ],
        out_shape=[
            jax.ShapeDtypeStruct((m, 128), F32),
            jax.ShapeDtypeStruct((2, 128), F32),
        ],
    )(e, neigh, bg2d, bwt, bb, cu, epsr)


def _norm_body(x_ref, sc_ref, sh_ref, en_ref, siglo_ref, sighi_ref):
    en = jnp.maximum(x_ref[...] * sc_ref[...] + sh_ref[...], 0.0)
    en_ref[...] = en
    sig = 1.0 / (1.0 + jnp.exp(-en))
    siglo_ref[...] = sig[:, :64]
    sighi_ref[...] = sig[:, 64:]


def _norm(epre, scale, shift):
    m = epre.shape[0]
    bm = 1000
    return pl.pallas_call(
        _norm_body,
        grid=(m // bm,),
        in_specs=[
            pl.BlockSpec((bm, 128), lambda i: (i, 0)),
            pl.BlockSpec((1, 128), lambda i: (0, 0)),
            pl.BlockSpec((1, 128), lambda i: (0, 0)),
        ],
        out_specs=[
            pl.BlockSpec((bm, 128), lambda i: (i, 0)),
            pl.BlockSpec((bm, 64), lambda i: (i, 0)),
            pl.BlockSpec((bm, 64), lambda i: (i, 0)),
        ],
        out_shape=[
            jax.ShapeDtypeStruct((m, 128), F32),
            jax.ShapeDtypeStruct((m, 64), F32),
            jax.ShapeDtypeStruct((m, 64), F32),
        ],
    )(epre, scale, shift)


def _node_body(ah_ref, dh_ref, pk_ref, ag_ref, fu_ref, eps_ref,
               g_ref, b_ref, out_ref):
    ah = ah_ref[...]
    ep = eps_ref[0, 0] + 1.0
    onehot = (ag_ref[...] == lax.broadcasted_iota(jnp.int32, (1, 16), 1)
              ).astype(F32)
    fu_rows = jnp.dot(onehot, fu_ref[...], preferred_element_type=F32)
    num = jnp.concatenate([pk_ref[0, :, :64], pk_ref[1, :, :64]], axis=-1)
    den = jnp.concatenate([pk_ref[0, :, 64:], pk_ref[1, :, 64:]], axis=-1)
    gated = num / (den + 1e-6)
    epa = ep * ah
    x = (jnp.maximum(epa + dh_ref[...], 0.0)
         + jnp.maximum(epa + fu_rows, 0.0)
         + gated)
    mu = jnp.mean(x, axis=0, keepdims=True)
    var = jnp.mean(x * x, axis=0, keepdims=True) - mu * mu
    y = (x - mu) * jax.lax.rsqrt(var + 1e-5) * g_ref[...] + b_ref[...]
    out_ref[...] = jnp.maximum(y, 0.0)


def _node(ah, dh, pk, ag2d, fu, epsr, g, b):
    n = ah.shape[0]
    return pl.pallas_call(
        _node_body,
        out_shape=jax.ShapeDtypeStruct((n, 128), F32),
    )(ah, dh, pk, ag2d, fu, epsr, g, b)


def _sc_neigh(ah, src, dst):
    m = src.shape[0]
    nw = NC * NS
    per_w = m // nw
    c = 104
    nb = 4
    n_rounds = per_w // (c * nb)
    tail = per_w - n_rounds * c * nb
    assert per_w * nw == m and c % 8 == 0 and tail % 8 == 0
    mesh = plsc.VectorSubcoreMesh(core_axis_name="c", subcore_axis_name="s")

    scratch = []
    for _ in range(nb):
        scratch += [pltpu.VMEM((c,), jnp.int32), pltpu.VMEM((c,), jnp.int32),
                    pltpu.VMEM((c, 128), F32)]
    scratch += [pltpu.SemaphoreType.DMA] * (4 * nb)

    @functools.partial(
        pl.kernel,
        out_type=jax.ShapeDtypeStruct((m, 128), F32),
        mesh=mesh,
        compiler_params=pltpu.CompilerParams(use_tc_tiling_on_sc=False),
        scratch_types=scratch,
    )
    def k(ah_hbm, src_hbm, dst_hbm, out_hbm, *scr):
        idx1 = [scr[3 * i] for i in range(nb)]
        idx2 = [scr[3 * i + 1] for i in range(nb)]
        rows = [scr[3 * i + 2] for i in range(nb)]
        sems = scr[3 * nb:]
        semld = sems[0:nb]
        semg1 = sems[nb:2 * nb]
        semg2 = sems[2 * nb:3 * nb]
        semst = sems[3 * nb:4 * nb]
        wid = lax.axis_index("s") * NC + lax.axis_index("c")
        base = wid * per_w

        def round_fn(r, carry):
            c0 = base + r * (c * nb)
            for i in range(nb):
                @pl.when(r > 0)
                def _(i=i):
                    pltpu.make_async_copy(
                        rows[i], out_hbm.at[pl.ds(0, c)], semst[i]).wait()
                pltpu.async_copy(
                    src_hbm.at[pl.ds(c0 + i * c, c)], idx1[i], semld[i])
                pltpu.async_copy(
                    dst_hbm.at[pl.ds(c0 + i * c, c)], idx2[i], semld[i])
            for i in range(nb):
                pltpu.make_async_copy(
                    src_hbm.at[pl.ds(0, c)], idx1[i], semld[i]).wait()
                pltpu.make_async_copy(
                    src_hbm.at[pl.ds(0, c)], idx2[i], semld[i]).wait()
                pltpu.async_copy(ah_hbm.at[idx1[i]], rows[i], semg1[i])
            for i in range(nb):
                pltpu.make_async_copy(
                    ah_hbm.at[idx1[i]], rows[i], semg1[i]).wait()
                pltpu.async_copy(ah_hbm.at[idx2[i]], rows[i], semg2[i],
                                 add=True)
            for i in range(nb):
                pltpu.make_async_copy(
                    ah_hbm.at[idx2[i]], rows[i], semg2[i]).wait()
                pltpu.async_copy(
                    rows[i], out_hbm.at[pl.ds(c0 + i * c, c)], semst[i])
            return carry

        lax.fori_loop(0, n_rounds, round_fn, 0)
        for i in range(nb):
            pltpu.make_async_copy(
                rows[i], out_hbm.at[pl.ds(0, c)], semst[i]).wait()
        if tail:
            tb = base + n_rounds * c * nb
            pltpu.sync_copy(src_hbm.at[pl.ds(tb, tail)],
                            idx1[0].at[pl.ds(0, tail)])
            pltpu.sync_copy(dst_hbm.at[pl.ds(tb, tail)],
                            idx2[0].at[pl.ds(0, tail)])
            rt = rows[0].at[pl.ds(0, tail)]
            pltpu.async_copy(ah_hbm.at[idx1[0].at[pl.ds(0, tail)]], rt,
                             semg1[0]).wait()
            pltpu.async_copy(ah_hbm.at[idx2[0].at[pl.ds(0, tail)]], rt,
                             semg2[0], add=True).wait()
            pltpu.sync_copy(rt, out_hbm.at[pl.ds(tb, tail)])

    return k(ah, src, dst)


def _sc_scatter(sigs_lo, sigs_hi, src, dst, ehs_lo, ehs_hi):
    m = src.shape[0]
    n = ehs_lo[0].shape[0]
    per_t = m // NS
    c = 64
    nb = 2
    n_rounds = per_t // (c * nb)
    tail = per_t - n_rounds * c * nb
    nz = (n // NS) & ~7
    rem = n - nz * NS
    assert per_t * NS == m and rem % 8 == 0 and tail % 8 == 0 and tail <= c
    mesh = plsc.VectorSubcoreMesh(core_axis_name="c", subcore_axis_name="s")

    scratch = []
    for _ in range(nb):
        scratch += [pltpu.VMEM((c,), jnp.int32), pltpu.VMEM((c,), jnp.int32),
                    pltpu.VMEM((c, 64), F32), pltpu.VMEM((c, 64), F32),
                    pltpu.VMEM((c, 128), F32), pltpu.VMEM((c, 128), F32)]
    scratch += [pltpu.VMEM((16,), jnp.int32), pltpu.VMEM((16,), jnp.int32)]
    scratch += [pltpu.VMEM_SHARED((n, 128), F32)]
    scratch += [pltpu.SemaphoreType.DMA] * (3 * nb)

    @functools.partial(
        pl.kernel,
        out_type=[jax.ShapeDtypeStruct((2, n, 128), F32) for _ in range(2)],
        mesh=mesh,
        compiler_params=pltpu.CompilerParams(use_tc_tiling_on_sc=False),
        scratch_types=scratch,
    )
    def k(siglo1_hbm, sighi1_hbm, siglo2_hbm, sighi2_hbm, src_hbm, dst_hbm,
          ehlo1_hbm, ehhi1_hbm, ehlo2_hbm, ehhi2_hbm,
          pk1_hbm, pk2_hbm, *scr):
        idx1 = [scr[6 * i] for i in range(nb)]
        idx2 = [scr[6 * i + 1] for i in range(nb)]
        ehj = [scr[6 * i + 2] for i in range(nb)]
        ehi = [scr[6 * i + 3] for i in range(nb)]
        payi = [scr[6 * i + 4] for i in range(nb)]
        payj = [scr[6 * i + 5] for i in range(nb)]
        p = 6 * nb
        idxt1, idxt2, accp = scr[p:p + 3]
        sems = scr[p + 3:]
        semld = sems[0:nb]
        semg = sems[nb:2 * nb]
        semsc = sems[2 * nb:3 * nb]
        cid = lax.axis_index("c")
        sid = lax.axis_index("s")
        r0 = sid * nz
        last = sid == NS - 1

        def zrow(r, carry):
            for kk in range(8):
                payi[0][r, pl.ds(16 * kk, 16)] = jnp.zeros((16,), F32)
            return carry

        def zero_acc():
            lax.fori_loop(0, c, zrow, 0)
            total = nz + rem
            off = 0
            while total > 0:
                cnt = min(c, total)
                pltpu.sync_copy(payi[0].at[pl.ds(0, cnt)],
                                accp.at[pl.ds(r0 + off, cnt)])
                off += cnt
                total -= cnt

        def mulrows(i, nrows):
            def row(r, carry2):
                for kk in range(4):
                    sl = pl.ds(16 * kk, 16)
                    s = payi[i][r, pl.ds(64 + 16 * kk, 16)]
                    payi[i][r, sl] = s * ehj[i][r, sl]
                    payj[i][r, sl] = s * ehi[i][r, sl]
                return carry2
            lax.fori_loop(0, nrows, row, 0, unroll=4)

        def phase(sig_hbm, eh_hbm, pk_out):
            plsc.subcore_barrier()
            zero_acc()
            plsc.subcore_barrier()
            base = sid * per_t

            def round_fn(r, carry):
                c0 = base + r * (c * nb)
                for i in range(nb):
                    pltpu.async_copy(
                        src_hbm.at[pl.ds(c0 + i * c, c)], idx1[i], semld[i])
                    pltpu.async_copy(
                        dst_hbm.at[pl.ds(c0 + i * c, c)], idx2[i], semld[i])
                    pltpu.async_copy(
                        sig_hbm.at[pl.ds(c0 + i * c, c)],
                        payi[i].at[:, pl.ds(64, 64)], semld[i])
                    pltpu.async_copy(
                        sig_hbm.at[pl.ds(c0 + i * c, c)],
                        payj[i].at[:, pl.ds(64, 64)], semld[i])
                for i in range(nb):
                    pltpu.make_async_copy(
                        src_hbm.at[pl.ds(0, c)], idx1[i], semld[i]).wait()
                    pltpu.make_async_copy(
                        src_hbm.at[pl.ds(0, c)], idx2[i], semld[i]).wait()
                    pltpu.make_async_copy(
                        sig_hbm.at[pl.ds(0, c)],
                        payi[i].at[:, pl.ds(64, 64)], semld[i]).wait()
                    pltpu.make_async_copy(
                        sig_hbm.at[pl.ds(0, c)],
                        payj[i].at[:, pl.ds(64, 64)], semld[i]).wait()
                    pltpu.async_copy(eh_hbm.at[idx2[i]], ehj[i], semg[i])
                    pltpu.async_copy(eh_hbm.at[idx1[i]], ehi[i], semg[i])
                for i in range(nb):
                    pltpu.make_async_copy(
                        eh_hbm.at[pl.ds(0, c)], ehj[i], semg[i]).wait()
                    pltpu.make_async_copy(
                        eh_hbm.at[pl.ds(0, c)], ehi[i], semg[i]).wait()
                    mulrows(i, c)
                    pltpu.async_copy(payi[i], accp.at[idx1[i]], semsc[i],
                                     add=True)
                    pltpu.make_async_copy(
                        payi[i], accp.at[idx1[i]], semsc[i]).wait()
                    pltpu.async_copy(payj[i], accp.at[idx2[i]], semsc[i],
                                     add=True)
                    pltpu.make_async_copy(
                        payj[i], accp.at[idx2[i]], semsc[i]).wait()
                return carry

            lax.fori_loop(0, n_rounds, round_fn, 0)
            if tail:
                tb = base + n_rounds * c * nb
                pltpu.sync_copy(src_hbm.at[pl.ds(tb, tail)], idxt1)
                pltpu.sync_copy(dst_hbm.at[pl.ds(tb, tail)], idxt2)
                pltpu.sync_copy(sig_hbm.at[pl.ds(tb, tail)],
                                payi[0].at[pl.ds(0, tail), pl.ds(64, 64)])
                pltpu.sync_copy(sig_hbm.at[pl.ds(tb, tail)],
                                payj[0].at[pl.ds(0, tail), pl.ds(64, 64)])
                pltpu.async_copy(eh_hbm.at[idxt2],
                                 ehj[0].at[pl.ds(0, tail)], semg[0]).wait()
                pltpu.async_copy(eh_hbm.at[idxt1],
                                 ehi[0].at[pl.ds(0, tail)], semg[0]).wait()
                mulrows(0, tail)
                pltpu.sync_copy(payi[0].at[pl.ds(0, tail)],
                                accp.at[idxt1], add=True)
                pltpu.sync_copy(payj[0].at[pl.ds(0, tail)],
                                accp.at[idxt2], add=True)
            plsc.subcore_barrier()

            def dump(cnt):
                pltpu.sync_copy(accp.at[pl.ds(r0, cnt)],
                                pk_out.at[pl.ds(r0, cnt)])

            @pl.when(jnp.logical_not(last))
            def _():
                dump(nz)

            @pl.when(last)
            def _():
                dump(nz + rem)

        @pl.when(cid == 0)
        def _():
            phase(siglo1_hbm, ehlo1_hbm, pk1_hbm.at[0])
            phase(siglo2_hbm, ehlo2_hbm, pk2_hbm.at[0])

        @pl.when(cid == 1)
        def _():
            phase(sighi1_hbm, ehhi1_hbm, pk1_hbm.at[1])
            phase(sighi2_hbm, ehhi2_hbm, pk2_hbm.at[1])

    return k(sigs_lo[0], sigs_hi[0], sigs_lo[1], sigs_hi[1], src, dst,
             ehs_lo[0], ehs_hi[0], ehs_lo[1], ehs_hi[1])


def kernel(h_atom, e_bond, h_atom2, e_bond2, u_global, bond_atoms, bond_graph,
           atom_graph, eps, A_W, A_b, B_W, B_b, C_W, C_b, D_W, D_b, E_W, E_b,
           F_W, F_b, bn_e_g, bn_e_b, bn_h_g, bn_h_b):
    m = e_bond.shape[0]
    src = bond_atoms[:, 0]
    dst = bond_atoms[:, 1]
    bg2d = bond_graph[:, None]
    ag2d = atom_graph[:, None]
    epsr = eps.reshape(1, 1)

    w_node = jnp.concatenate([A_W.T, D_W.T, E_W.T], axis=1)
    b_node = jnp.concatenate([A_b, D_b, E_b])[None, :]
    w_u = jnp.concatenate([C_W.T, F_W.T], axis=1)
    b_u = jnp.concatenate([C_b, F_b])[None, :]
    bwt = B_W.T
    bb = B_b[None, :]

    cu, fu = _u_lin(u_global, w_u, b_u)

    def edge_stage(h, e):
        ah, dh, ehlo, ehhi = _node_lin(h, w_node, b_node)
        neigh = _sc_neigh(ah, src, dst)
        epre, stats = _edge(e, neigh, bg2d, bwt, bb, cu, epsr)
        mu = stats[0] / m
        var = stats[1] / m - mu * mu
        scale = bn_e_g * lax.rsqrt(var + 1e-5)
        shift = bn_e_b - mu * scale
        e_new, siglo, sighi = _norm(epre, scale[None, :], shift[None, :])
        return ah, dh, ehlo, ehhi, e_new, siglo, sighi

    ah1, dh1, ehlo1, ehhi1, e1, siglo1, sighi1 = edge_stage(h_atom, e_bond)
    ah2, dh2, ehlo2, ehhi2, e2, siglo2, sighi2 = edge_stage(h_atom2, e_bond2)

    pk1, pk2 = _sc_scatter(
        (siglo1, siglo2), (sighi1, sighi2), src, dst,
        (ehlo1, ehlo2), (ehhi1, ehhi2))

    h1 = _node(ah1, dh1, pk1, ag2d, fu, epsr,
               bn_h_g[None, :], bn_h_b[None, :])
    h2 = _node(ah2, dh2, pk2, ag2d, fu, epsr,
               bn_h_g[None, :], bn_h_b[None, :])
    return (h1, e1, h2, e2)


# speedup vs baseline: 1.1118x; 1.0896x over previous
import functools

import jax
import jax.numpy as jnp
from jax import lax
from jax.experimental import pallas as pl
from jax.experimental.pallas import tpu as pltpu
from jax.experimental.pallas import tpu_sc as plsc

F32 = jnp.float32
NC, NS = 2, 16


def _node_lin_body(h_ref, w_ref, b_ref, ah_ref, dh_ref, ehlo_ref, ehhi_ref):
    y = jnp.dot(h_ref[...], w_ref[...], preferred_element_type=F32) + b_ref[...]
    ah_ref[...] = y[:, :128]
    dh_ref[...] = y[:, 128:256]
    ehlo_ref[...] = y[:, 256:320]
    ehhi_ref[...] = y[:, 320:384]


def _node_lin(h, w_cat, b_cat):
    n = h.shape[0]
    bn = 2000
    return pl.pallas_call(
        _node_lin_body,
        grid=(n // bn,),
        in_specs=[
            pl.BlockSpec((bn, 128), lambda i: (i, 0)),
            pl.BlockSpec((128, 384), lambda i: (0, 0)),
            pl.BlockSpec((1, 384), lambda i: (0, 0)),
        ],
        out_specs=[
            pl.BlockSpec((bn, 128), lambda i: (i, 0)),
            pl.BlockSpec((bn, 128), lambda i: (i, 0)),
            pl.BlockSpec((bn, 64), lambda i: (i, 0)),
            pl.BlockSpec((bn, 64), lambda i: (i, 0)),
        ],
        out_shape=[
            jax.ShapeDtypeStruct((n, 128), F32),
            jax.ShapeDtypeStruct((n, 128), F32),
            jax.ShapeDtypeStruct((n, 64), F32),
            jax.ShapeDtypeStruct((n, 64), F32),
        ],
    )(h, w_cat, b_cat)


def _u_lin_body(u_ref, w_ref, b_ref, cu_ref, fu_ref):
    y = jnp.dot(u_ref[...], w_ref[...], preferred_element_type=F32) + b_ref[...]
    cu_ref[...] = y[:, :128]
    fu_ref[...] = y[:, 128:256]


def _u_lin(u, w_cat, b_cat):
    g = u.shape[0]
    return pl.pallas_call(
        _u_lin_body,
        out_shape=[
            jax.ShapeDtypeStruct((g, 128), F32),
            jax.ShapeDtypeStruct((g, 128), F32),
        ],
    )(u, w_cat, b_cat)


def _edge_body(e_ref, nb_ref, bg_ref, bwt_ref, bb_ref, cu_ref, eps_ref,
               epre_ref, stats_ref):
    i = pl.program_id(0)
    e = e_ref[...]
    be = jnp.dot(e, bwt_ref[...], preferred_element_type=F32) + bb_ref[...]
    ep = eps_ref[0, 0] + 1.0
    onehot = (bg_ref[...] == lax.broadcasted_iota(jnp.int32, (1, 16), 1)
              ).astype(F32)
    cu_rows = jnp.dot(onehot, cu_ref[...], preferred_element_type=F32)
    epb = ep * be
    x = (jnp.maximum(epb + nb_ref[...], 0.0)
         + jnp.maximum(epb + be, 0.0)
         + jnp.maximum(epb + cu_rows, 0.0))
    epre_ref[...] = x
    s1 = jnp.sum(x, axis=0, keepdims=True)
    s2 = jnp.sum(x * x, axis=0, keepdims=True)
    blk = jnp.concatenate([s1, s2], axis=0)

    @pl.when(i == 0)
    def _():
        stats_ref[...] = blk

    @pl.when(i > 0)
    def _():
        stats_ref[...] += blk


def _edge(e, neigh, bg2d, bwt, bb, cu, epsr):
    m = e.shape[0]
    bm = 1000
    return pl.pallas_call(
        _edge_body,
        grid=(m // bm,),
        in_specs=[
            pl.BlockSpec((bm, 128), lambda i: (i, 0)),
            pl.BlockSpec((bm, 128), lambda i: (i, 0)),
            pl.BlockSpec((bm, 1), lambda i: (i, 0)),
            pl.BlockSpec((128, 128), lambda i: (0, 0)),
            pl.BlockSpec((1, 128), lambda i: (0, 0)),
            pl.BlockSpec((16, 128), lambda i: (0, 0)),
            pl.BlockSpec((1, 1), lambda i: (0, 0)),
        ],
        out_specs=[
            pl.BlockSpec((bm, 128), lambda i: (i, 0)),
            pl.BlockSpec((2, 128), lambda i: (0, 0)),
        ],
        out_shape=[
            jax.ShapeDtypeStruct((m, 128), F32),
            jax.ShapeDtypeStruct((2, 128), F32),
        ],
    )(e, neigh, bg2d, bwt, bb, cu, epsr)


def _norm_body(x_ref, sc_ref, sh_ref, en_ref, siglo_ref, sighi_ref):
    en = jnp.maximum(x_ref[...] * sc_ref[...] + sh_ref[...], 0.0)
    en_ref[...] = en
    sig = 1.0 / (1.0 + jnp.exp(-en))
    siglo_ref[...] = sig[:, :64]
    sighi_ref[...] = sig[:, 64:]


def _norm(epre, scale, shift):
    m = epre.shape[0]
    bm = 1000
    return pl.pallas_call(
        _norm_body,
        grid=(m // bm,),
        in_specs=[
            pl.BlockSpec((bm, 128), lambda i: (i, 0)),
            pl.BlockSpec((1, 128), lambda i: (0, 0)),
            pl.BlockSpec((1, 128), lambda i: (0, 0)),
        ],
        out_specs=[
            pl.BlockSpec((bm, 128), lambda i: (i, 0)),
            pl.BlockSpec((bm, 64), lambda i: (i, 0)),
            pl.BlockSpec((bm, 64), lambda i: (i, 0)),
        ],
        out_shape=[
            jax.ShapeDtypeStruct((m, 128), F32),
            jax.ShapeDtypeStruct((m, 64), F32),
            jax.ShapeDtypeStruct((m, 64), F32),
        ],
    )(epre, scale, shift)


def _node_body(ah_ref, dh_ref, pk_ref, ag_ref, fu_ref, eps_ref,
               g_ref, b_ref, out_ref):
    ah = ah_ref[...]
    ep = eps_ref[0, 0] + 1.0
    onehot = (ag_ref[...] == lax.broadcasted_iota(jnp.int32, (1, 16), 1)
              ).astype(F32)
    fu_rows = jnp.dot(onehot, fu_ref[...], preferred_element_type=F32)
    num = jnp.concatenate([pk_ref[0, :, :64], pk_ref[1, :, :64]], axis=-1)
    den = jnp.concatenate([pk_ref[0, :, 64:], pk_ref[1, :, 64:]], axis=-1)
    gated = num / (den + 1e-6)
    epa = ep * ah
    x = (jnp.maximum(epa + dh_ref[...], 0.0)
         + jnp.maximum(epa + fu_rows, 0.0)
         + gated)
    mu = jnp.mean(x, axis=0, keepdims=True)
    var = jnp.mean(x * x, axis=0, keepdims=True) - mu * mu
    y = (x - mu) * jax.lax.rsqrt(var + 1e-5) * g_ref[...] + b_ref[...]
    out_ref[...] = jnp.maximum(y, 0.0)


def _node(ah, dh, pk, ag2d, fu, epsr, g, b):
    n = ah.shape[0]
    return pl.pallas_call(
        _node_body,
        out_shape=jax.ShapeDtypeStruct((n, 128), F32),
    )(ah, dh, pk, ag2d, fu, epsr, g, b)


def _sc_neigh(ah, src, dst):
    m = src.shape[0]
    nw = NC * NS
    per_w = m // nw
    c = 104
    nb = 4
    n_rounds = per_w // (c * nb)
    tail = per_w - n_rounds * c * nb
    assert per_w * nw == m and c % 8 == 0 and tail % 8 == 0
    mesh = plsc.VectorSubcoreMesh(core_axis_name="c", subcore_axis_name="s")

    scratch = []
    for _ in range(nb):
        scratch += [pltpu.VMEM((c,), jnp.int32), pltpu.VMEM((c,), jnp.int32),
                    pltpu.VMEM((c, 128), F32)]
    scratch += [pltpu.SemaphoreType.DMA] * (4 * nb)

    @functools.partial(
        pl.kernel,
        out_type=jax.ShapeDtypeStruct((m, 128), F32),
        mesh=mesh,
        compiler_params=pltpu.CompilerParams(use_tc_tiling_on_sc=False),
        scratch_types=scratch,
    )
    def k(ah_hbm, src_hbm, dst_hbm, out_hbm, *scr):
        idx1 = [scr[3 * i] for i in range(nb)]
        idx2 = [scr[3 * i + 1] for i in range(nb)]
        rows = [scr[3 * i + 2] for i in range(nb)]
        sems = scr[3 * nb:]
        semld = sems[0:nb]
        semg1 = sems[nb:2 * nb]
        semg2 = sems[2 * nb:3 * nb]
        semst = sems[3 * nb:4 * nb]
        wid = lax.axis_index("s") * NC + lax.axis_index("c")
        base = wid * per_w

        def round_fn(r, carry):
            c0 = base + r * (c * nb)
            for i in range(nb):
                @pl.when(r > 0)
                def _(i=i):
                    pltpu.make_async_copy(
                        rows[i], out_hbm.at[pl.ds(0, c)], semst[i]).wait()
                pltpu.async_copy(
                    src_hbm.at[pl.ds(c0 + i * c, c)], idx1[i], semld[i])
                pltpu.async_copy(
                    dst_hbm.at[pl.ds(c0 + i * c, c)], idx2[i], semld[i])
            for i in range(nb):
                pltpu.make_async_copy(
                    src_hbm.at[pl.ds(0, c)], idx1[i], semld[i]).wait()
                pltpu.make_async_copy(
                    src_hbm.at[pl.ds(0, c)], idx2[i], semld[i]).wait()
                pltpu.async_copy(ah_hbm.at[idx1[i]], rows[i], semg1[i])
            for i in range(nb):
                pltpu.make_async_copy(
                    ah_hbm.at[idx1[i]], rows[i], semg1[i]).wait()
                pltpu.async_copy(ah_hbm.at[idx2[i]], rows[i], semg2[i],
                                 add=True)
            for i in range(nb):
                pltpu.make_async_copy(
                    ah_hbm.at[idx2[i]], rows[i], semg2[i]).wait()
                pltpu.async_copy(
                    rows[i], out_hbm.at[pl.ds(c0 + i * c, c)], semst[i])
            return carry

        lax.fori_loop(0, n_rounds, round_fn, 0)
        for i in range(nb):
            pltpu.make_async_copy(
                rows[i], out_hbm.at[pl.ds(0, c)], semst[i]).wait()
        if tail:
            tb = base + n_rounds * c * nb
            pltpu.sync_copy(src_hbm.at[pl.ds(tb, tail)],
                            idx1[0].at[pl.ds(0, tail)])
            pltpu.sync_copy(dst_hbm.at[pl.ds(tb, tail)],
                            idx2[0].at[pl.ds(0, tail)])
            rt = rows[0].at[pl.ds(0, tail)]
            pltpu.async_copy(ah_hbm.at[idx1[0].at[pl.ds(0, tail)]], rt,
                             semg1[0]).wait()
            pltpu.async_copy(ah_hbm.at[idx2[0].at[pl.ds(0, tail)]], rt,
                             semg2[0], add=True).wait()
            pltpu.sync_copy(rt, out_hbm.at[pl.ds(tb, tail)])

    return k(ah, src, dst)


def _sc_scatter(sigs_lo, sigs_hi, src, dst, ehs_lo, ehs_hi):
    m = src.shape[0]
    n = ehs_lo[0].shape[0]
    per_t = m // NS
    c = 64
    nb = 2
    n_rounds = per_t // (c * nb)
    tail = per_t - n_rounds * c * nb
    nz = (n // NS) & ~7
    rem = n - nz * NS
    assert per_t * NS == m and rem % 8 == 0 and tail % 8 == 0 and tail <= c
    mesh = plsc.VectorSubcoreMesh(core_axis_name="c", subcore_axis_name="s")

    scratch = []
    for _ in range(nb):
        scratch += [pltpu.VMEM((c,), jnp.int32), pltpu.VMEM((c,), jnp.int32),
                    pltpu.VMEM((c, 64), F32), pltpu.VMEM((c, 64), F32),
                    pltpu.VMEM((c, 128), F32), pltpu.VMEM((c, 128), F32)]
    scratch += [pltpu.VMEM((16,), jnp.int32), pltpu.VMEM((16,), jnp.int32)]
    scratch += [pltpu.VMEM_SHARED((n, 128), F32)]
    scratch += [pltpu.SemaphoreType.DMA] * (3 * nb)

    @functools.partial(
        pl.kernel,
        out_type=[jax.ShapeDtypeStruct((2, n, 128), F32) for _ in range(2)],
        mesh=mesh,
        compiler_params=pltpu.CompilerParams(use_tc_tiling_on_sc=False),
        scratch_types=scratch,
    )
    def k(siglo1_hbm, sighi1_hbm, siglo2_hbm, sighi2_hbm, src_hbm, dst_hbm,
          ehlo1_hbm, ehhi1_hbm, ehlo2_hbm, ehhi2_hbm,
          pk1_hbm, pk2_hbm, *scr):
        idx1 = [scr[6 * i] for i in range(nb)]
        idx2 = [scr[6 * i + 1] for i in range(nb)]
        ehj = [scr[6 * i + 2] for i in range(nb)]
        ehi = [scr[6 * i + 3] for i in range(nb)]
        payi = [scr[6 * i + 4] for i in range(nb)]
        payj = [scr[6 * i + 5] for i in range(nb)]
        p = 6 * nb
        idxt1, idxt2, accp = scr[p:p + 3]
        sems = scr[p + 3:]
        semld = sems[0:nb]
        semg = sems[nb:2 * nb]
        semsc = sems[2 * nb:3 * nb]
        cid = lax.axis_index("c")
        sid = lax.axis_index("s")
        r0 = sid * nz
        last = sid == NS - 1

        def zrow(r, carry):
            for kk in range(8):
                payi[0][r, pl.ds(16 * kk, 16)] = jnp.zeros((16,), F32)
            return carry

        def zero_acc():
            lax.fori_loop(0, c, zrow, 0)
            total = nz + rem
            off = 0
            while total > 0:
                cnt = min(c, total)
                pltpu.sync_copy(payi[0].at[pl.ds(0, cnt)],
                                accp.at[pl.ds(r0 + off, cnt)])
                off += cnt
                total -= cnt

        def mulrows(i, nrows):
            def row(r, carry2):
                for kk in range(4):
                    sl = pl.ds(16 * kk, 16)
                    s = payi[i][r, pl.ds(64 + 16 * kk, 16)]
                    payi[i][r, sl] = s * ehj[i][r, sl]
                    payj[i][r, sl] = s * ehi[i][r, sl]
                return carry2
            lax.fori_loop(0, nrows, row, 0, unroll=4)

        def phase(sig_hbm, eh_hbm, pk_out):
            plsc.subcore_barrier()
            zero_acc()
            plsc.subcore_barrier()
            base = sid * per_t

            def round_fn(r, carry):
                c0 = base + r * (c * nb)
                for i in range(nb):
                    @pl.when(r > 0)
                    def _(i=i):
                        pltpu.make_async_copy(
                            payi[i], accp.at[idx1[i]], semsc[i]).wait()
                        pltpu.make_async_copy(
                            payj[i], accp.at[idx2[i]], semsc[i]).wait()
                    pltpu.async_copy(
                        src_hbm.at[pl.ds(c0 + i * c, c)], idx1[i], semld[i])
                    pltpu.async_copy(
                        dst_hbm.at[pl.ds(c0 + i * c, c)], idx2[i], semld[i])
                    pltpu.async_copy(
                        sig_hbm.at[pl.ds(c0 + i * c, c)],
                        payi[i].at[:, pl.ds(64, 64)], semld[i])
                    pltpu.async_copy(
                        sig_hbm.at[pl.ds(c0 + i * c, c)],
                        payj[i].at[:, pl.ds(64, 64)], semld[i])
                for i in range(nb):
                    pltpu.make_async_copy(
                        src_hbm.at[pl.ds(0, c)], idx1[i], semld[i]).wait()
                    pltpu.make_async_copy(
                        src_hbm.at[pl.ds(0, c)], idx2[i], semld[i]).wait()
                    pltpu.make_async_copy(
                        sig_hbm.at[pl.ds(0, c)],
                        payi[i].at[:, pl.ds(64, 64)], semld[i]).wait()
                    pltpu.make_async_copy(
                        sig_hbm.at[pl.ds(0, c)],
                        payj[i].at[:, pl.ds(64, 64)], semld[i]).wait()
                    pltpu.async_copy(eh_hbm.at[idx2[i]], ehj[i], semg[i])
                    pltpu.async_copy(eh_hbm.at[idx1[i]], ehi[i], semg[i])
                for i in range(nb):
                    pltpu.make_async_copy(
                        eh_hbm.at[pl.ds(0, c)], ehj[i], semg[i]).wait()
                    pltpu.make_async_copy(
                        eh_hbm.at[pl.ds(0, c)], ehi[i], semg[i]).wait()
                    mulrows(i, c)
                    pltpu.async_copy(payi[i], accp.at[idx1[i]], semsc[i],
                                     add=True)
                    pltpu.async_copy(payj[i], accp.at[idx2[i]], semsc[i],
                                     add=True)
                return carry

            lax.fori_loop(0, n_rounds, round_fn, 0)
            for i in range(nb):
                pltpu.make_async_copy(
                    payi[i], accp.at[idx1[i]], semsc[i]).wait()
                pltpu.make_async_copy(
                    payj[i], accp.at[idx2[i]], semsc[i]).wait()
            if tail:
                tb = base + n_rounds * c * nb
                pltpu.sync_copy(src_hbm.at[pl.ds(tb, tail)], idxt1)
                pltpu.sync_copy(dst_hbm.at[pl.ds(tb, tail)], idxt2)
                pltpu.sync_copy(sig_hbm.at[pl.ds(tb, tail)],
                                payi[0].at[pl.ds(0, tail), pl.ds(64, 64)])
                pltpu.sync_copy(sig_hbm.at[pl.ds(tb, tail)],
                                payj[0].at[pl.ds(0, tail), pl.ds(64, 64)])
                pltpu.async_copy(eh_hbm.at[idxt2],
                                 ehj[0].at[pl.ds(0, tail)], semg[0]).wait()
                pltpu.async_copy(eh_hbm.at[idxt1],
                                 ehi[0].at[pl.ds(0, tail)], semg[0]).wait()
                mulrows(0, tail)
                pltpu.sync_copy(payi[0].at[pl.ds(0, tail)],
                                accp.at[idxt1], add=True)
                pltpu.sync_copy(payj[0].at[pl.ds(0, tail)],
                                accp.at[idxt2], add=True)
            plsc.subcore_barrier()

            def dump(cnt):
                pltpu.sync_copy(accp.at[pl.ds(r0, cnt)],
                                pk_out.at[pl.ds(r0, cnt)])

            @pl.when(jnp.logical_not(last))
            def _():
                dump(nz)

            @pl.when(last)
            def _():
                dump(nz + rem)

        @pl.when(cid == 0)
        def _():
            phase(siglo1_hbm, ehlo1_hbm, pk1_hbm.at[0])
            phase(siglo2_hbm, ehlo2_hbm, pk2_hbm.at[0])

        @pl.when(cid == 1)
        def _():
            phase(sighi1_hbm, ehhi1_hbm, pk1_hbm.at[1])
            phase(sighi2_hbm, ehhi2_hbm, pk2_hbm.at[1])

    return k(sigs_lo[0], sigs_hi[0], sigs_lo[1], sigs_hi[1], src, dst,
             ehs_lo[0], ehs_hi[0], ehs_lo[1], ehs_hi[1])


def kernel(h_atom, e_bond, h_atom2, e_bond2, u_global, bond_atoms, bond_graph,
           atom_graph, eps, A_W, A_b, B_W, B_b, C_W, C_b, D_W, D_b, E_W, E_b,
           F_W, F_b, bn_e_g, bn_e_b, bn_h_g, bn_h_b):
    m = e_bond.shape[0]
    src = bond_atoms[:, 0]
    dst = bond_atoms[:, 1]
    bg2d = bond_graph[:, None]
    ag2d = atom_graph[:, None]
    epsr = eps.reshape(1, 1)

    w_node = jnp.concatenate([A_W.T, D_W.T, E_W.T], axis=1)
    b_node = jnp.concatenate([A_b, D_b, E_b])[None, :]
    w_u = jnp.concatenate([C_W.T, F_W.T], axis=1)
    b_u = jnp.concatenate([C_b, F_b])[None, :]
    bwt = B_W.T
    bb = B_b[None, :]

    cu, fu = _u_lin(u_global, w_u, b_u)

    def edge_stage(h, e):
        ah, dh, ehlo, ehhi = _node_lin(h, w_node, b_node)
        neigh = _sc_neigh(ah, src, dst)
        epre, stats = _edge(e, neigh, bg2d, bwt, bb, cu, epsr)
        mu = stats[0] / m
        var = stats[1] / m - mu * mu
        scale = bn_e_g * lax.rsqrt(var + 1e-5)
        shift = bn_e_b - mu * scale
        e_new, siglo, sighi = _norm(epre, scale[None, :], shift[None, :])
        return ah, dh, ehlo, ehhi, e_new, siglo, sighi

    ah1, dh1, ehlo1, ehhi1, e1, siglo1, sighi1 = edge_stage(h_atom, e_bond)
    ah2, dh2, ehlo2, ehhi2, e2, siglo2, sighi2 = edge_stage(h_atom2, e_bond2)

    pk1, pk2 = _sc_scatter(
        (siglo1, siglo2), (sighi1, sighi2), src, dst,
        (ehlo1, ehlo2), (ehhi1, ehhi2))

    h1 = _node(ah1, dh1, pk1, ag2d, fu, epsr,
               bn_h_g[None, :], bn_h_b[None, :])
    h2 = _node(ah2, dh2, pk2, ag2d, fu, epsr,
               bn_h_g[None, :], bn_h_b[None, :])
    return (h1, e1, h2, e2)
